# baseline (speedup 1.0000x reference)
import functools

import jax
import jax.numpy as jnp
from jax import lax
from jax.experimental import pallas as pl
from jax.experimental.pallas import tpu as pltpu
from jax.experimental.pallas import tpu_sc as plsc

VOCAB = 1_000_000
H = 64
HH = H // 2
B = 16384
L = 20

NC = 2
NS = 16
NW = NC * NS
BPW = B // NW
CB = 32
NCH = BPW // CB
RPC = CB * L
NG = RPC // 128


def _make_body(with_prev):
    def body(*refs):
        if with_prev:
            (ctx_hbm, u_hbm, v_hbm, prev_hbm, out_hbm,
             vidx, urows, vrows, tbufa, tbufb, pbuf, sbuf, gsem, usem) = refs
        else:
            (ctx_hbm, u_hbm, v_hbm, out_hbm,
             vidx, urows, vrows, tbufa, tbufb, pbuf, sbuf, gsem, usem) = refs
        wid = lax.axis_index("s") * NC + lax.axis_index("c")
        iota16 = lax.iota(jnp.int32, 16)

        def fire(g, slot):
            base = pl.multiple_of(wid * BPW + g * CB, CB)
            off = pl.multiple_of((wid * BPW + g * CB) * L, RPC)
            pltpu.sync_copy(ctx_hbm.at[pl.ds(off, RPC)], vidx.at[slot])
            pltpu.async_copy(u_hbm.at[pl.ds(base, CB)], urows.at[slot],
                             usem.at[slot])
            for j in range(NG):
                pltpu.async_copy(v_hbm.at[vidx.at[slot, pl.ds(j * 128, 128)]],
                                 vrows.at[slot, pl.ds(j * 128, 128)],
                                 gsem.at[slot])

        def wait_gathers(slot):
            base = pl.multiple_of(0, CB)
            pltpu.make_async_copy(u_hbm.at[pl.ds(base, CB)], urows.at[slot],
                                  usem.at[slot]).wait()
            for j in range(NG):
                pltpu.make_async_copy(
                    v_hbm.at[vidx.at[slot, pl.ds(j * 128, 128)]],
                    vrows.at[slot, pl.ds(j * 128, 128)],
                    gsem.at[slot]).wait()

        def compute(g, slot):
            def hsum16(tb):
                gs = [plsc.load_gather(tb, [iota16 * 16 + k])
                      for k in range(16)]
                while len(gs) > 1:
                    gs = [gs[i] + gs[i + 1] for i in range(0, len(gs), 2)]
                return gs[0]

            evens = iota16 * 2

            def bbody(b, carry):
                bsplat = jnp.full((16,), 0, jnp.int32) + b
                ue = plsc.load_gather(urows.at[slot], [bsplat, evens])
                uo = plsc.load_gather(urows.at[slot], [bsplat, evens + 1])

                def part(r):
                    ve, vo = plsc.unpack(vrows[slot, r, pl.ds(0, 2 * 16)],
                                         format=plsc.PackFormat.INTERLEAVED)
                    return ve * ue + vo * uo

                for l in range(16):
                    tbufa[pl.ds(l * 16, 16)] = part(b * L + l)
                for l in range(16, L):
                    tbufb[pl.ds((l - 16) * 16, 16)] = part(b * L + l)
                sbuf[slot, pl.ds(b * L, 16)] = hsum16(tbufa)
                sbuf[slot, pl.ds(b * L + 16, 16)] = hsum16(tbufb)
                return carry

            lax.fori_loop(0, CB, bbody, 0)
            base = pl.multiple_of((wid * BPW + g * CB) * L, RPC)
            if with_prev:
                pltpu.sync_copy(prev_hbm.at[pl.ds(base, RPC)], pbuf)
                for k in range(RPC // 16):
                    sbuf[slot, pl.ds(k * 16, 16)] = (
                        sbuf[slot, pl.ds(k * 16, 16)] + pbuf[pl.ds(k * 16, 16)])
            pltpu.sync_copy(sbuf.at[slot, pl.ds(0, RPC)],
                            out_hbm.at[pl.ds(base, RPC)])

        fire(0, 0)

        def pair(i, carry):
            for s in (0, 1):
                g = i * 2 + s

                @pl.when(g + 1 < NCH)
                def _():
                    fire(g + 1, (s + 1) % 2)

                wait_gathers(s)
                compute(g, s)
            return carry

        lax.fori_loop(0, NCH // 2, pair, 0)

    return body


def _sc_scores_half(with_prev, *args):
    mesh = plsc.VectorSubcoreMesh(core_axis_name="c", subcore_axis_name="s",
                                  num_cores=NC, num_subcores=NS)
    return pl.kernel(
        _make_body(with_prev),
        out_type=jax.ShapeDtypeStruct((B * L,), jnp.float32),
        mesh=mesh,
        scratch_types=[
            pltpu.VMEM((2, RPC), jnp.int32),
            pltpu.VMEM((2, CB, HH), jnp.float32),
            pltpu.VMEM((2, RPC + 16, HH), jnp.bfloat16),
            pltpu.VMEM((256,), jnp.float32),
            pltpu.VMEM((256,), jnp.float32),
            pltpu.VMEM((RPC,), jnp.float32),
            pltpu.VMEM((2, RPC + 32), jnp.float32),
            pltpu.SemaphoreType.DMA((2,)),
            pltpu.SemaphoreType.DMA((2,)),
        ],
        compiler_params=pltpu.CompilerParams(needs_layout_passes=False,
                                             use_tc_tiling_on_sc=False),
    )(*args)


def _loss_body(s_ref, lab_ref, m_ref, out_ref):
    s = s_ref[...]
    lab = lab_ref[...]
    m = m_ref[...]
    per = jnp.maximum(s, 0.0) - s * lab + jnp.log1p(jnp.exp(-jnp.abs(s)))
    num = jnp.sum(per * m)
    den = jnp.maximum(jnp.sum(m), 1.0)
    out_ref[0, 0] = num / den


def _tc_loss(scores2d, label2d, mask2d):
    return pl.pallas_call(
        _loss_body,
        out_shape=jax.ShapeDtypeStruct((1, 1), jnp.float32),
        out_specs=pl.BlockSpec(memory_space=pltpu.SMEM),
    )(scores2d, label2d, mask2d)


def kernel(center, context_neg, label, mask, U, V):
    ctx_flat = context_neg.reshape(B * L)
    u_pre = jnp.take(U, center[:, 0], axis=0)
    va = V[:, :HH].astype(jnp.bfloat16)
    vb = V[:, HH:].astype(jnp.bfloat16)
    sa = _sc_scores_half(False, ctx_flat, u_pre[:, :HH], va)
    scores = _sc_scores_half(True, ctx_flat, u_pre[:, HH:], vb, sa)
    scores2d = scores.reshape(B * L // 128, 128)
    label2d = label.reshape(B * L // 128, 128)
    mask2d = mask.reshape(B * L // 128, 128)
    return _tc_loss(scores2d, label2d, mask2d).reshape(())


# speedup vs baseline: 1.6513x; 1.6513x over previous
import functools

import jax
import jax.numpy as jnp
from jax import lax
from jax.experimental import pallas as pl
from jax.experimental.pallas import tpu as pltpu
from jax.experimental.pallas import tpu_sc as plsc

VOCAB = 1_000_000
H = 64
B = 16384
L = 20

NC = 2
NS = 16
NW = NC * NS
BPW = B // NW
CB = 32
NCH = BPW // CB
RPC = CB * L
NG = RPC // 128


def _sc_scores_body(ctx_hbm, u_hbm, v_hbm, out_hbm,
                    vidx, urows, vrows, tbufa, tbufb, sbuf, gsem, usem):
    wid = lax.axis_index("s") * NC + lax.axis_index("c")
    iota16 = lax.iota(jnp.int32, 16)

    def fire(g, slot):
        base = pl.multiple_of(wid * BPW + g * CB, CB)
        off = pl.multiple_of((wid * BPW + g * CB) * L, RPC)
        pltpu.sync_copy(ctx_hbm.at[pl.ds(off, RPC)], vidx.at[slot])
        pltpu.async_copy(u_hbm.at[pl.ds(base, CB)], urows.at[slot],
                         usem.at[slot])
        for j in range(NG):
            pltpu.async_copy(v_hbm.at[vidx.at[slot, pl.ds(j * 128, 128)]],
                             vrows.at[slot, pl.ds(j * 128, 128)],
                             gsem.at[slot])

    def wait_gathers(slot):
        base = pl.multiple_of(0, CB)
        pltpu.make_async_copy(u_hbm.at[pl.ds(base, CB)], urows.at[slot],
                              usem.at[slot]).wait()
        for j in range(NG):
            pltpu.make_async_copy(v_hbm.at[vidx.at[slot, pl.ds(j * 128, 128)]],
                                  vrows.at[slot, pl.ds(j * 128, 128)],
                                  gsem.at[slot]).wait()

    def compute(g, slot):
        def hsum16(tb):
            gs = [plsc.load_gather(tb, [iota16 * 16 + k]) for k in range(16)]
            while len(gs) > 1:
                gs = [gs[i] + gs[i + 1] for i in range(0, len(gs), 2)]
            return gs[0]

        evens = iota16 * 2

        def bbody(b, carry):
            bsplat = jnp.full((16,), 0, jnp.int32) + b
            ue0 = plsc.load_gather(urows.at[slot], [bsplat, evens])
            uo0 = plsc.load_gather(urows.at[slot], [bsplat, evens + 1])
            ue1 = plsc.load_gather(urows.at[slot], [bsplat, evens + 32])
            uo1 = plsc.load_gather(urows.at[slot], [bsplat, evens + 33])

            def part(r):
                v0e, v0o = plsc.unpack(vrows[slot, r, pl.ds(0, 32)],
                                       format=plsc.PackFormat.INTERLEAVED)
                v1e, v1o = plsc.unpack(vrows[slot, r, pl.ds(32, 32)],
                                       format=plsc.PackFormat.INTERLEAVED)
                return v0e * ue0 + v0o * uo0 + v1e * ue1 + v1o * uo1

            for l in range(16):
                tbufa[pl.ds(l * 16, 16)] = part(b * L + l)
            for l in range(16, L):
                tbufb[pl.ds((l - 16) * 16, 16)] = part(b * L + l)
            sbuf[slot, pl.ds(b * L, 16)] = hsum16(tbufa)
            sbuf[slot, pl.ds(b * L + 16, 16)] = hsum16(tbufb)
            return carry

        lax.fori_loop(0, CB, bbody, 0)
        base = pl.multiple_of((wid * BPW + g * CB) * L, RPC)
        pltpu.sync_copy(sbuf.at[slot, pl.ds(0, RPC)],
                        out_hbm.at[pl.ds(base, RPC)])

    fire(0, 0)

    def pair(i, carry):
        for s in (0, 1):
            g = i * 2 + s

            @pl.when(g + 1 < NCH)
            def _():
                fire(g + 1, (s + 1) % 2)

            wait_gathers(s)
            compute(g, s)
        return carry

    lax.fori_loop(0, NCH // 2, pair, 0)


def _sc_scores(ctx_flat, u_pre, v16):
    mesh = plsc.VectorSubcoreMesh(core_axis_name="c", subcore_axis_name="s",
                                  num_cores=NC, num_subcores=NS)
    return pl.kernel(
        _sc_scores_body,
        out_type=jax.ShapeDtypeStruct((B * L,), jnp.float32),
        mesh=mesh,
        scratch_types=[
            pltpu.VMEM((2, RPC), jnp.int32),
            pltpu.VMEM((2, CB, H), jnp.float32),
            pltpu.VMEM((2, RPC + 16, H), jnp.bfloat16),
            pltpu.VMEM((256,), jnp.float32),
            pltpu.VMEM((256,), jnp.float32),
            pltpu.VMEM((2, RPC + 32), jnp.float32),
            pltpu.SemaphoreType.DMA((2,)),
            pltpu.SemaphoreType.DMA((2,)),
        ],
        compiler_params=pltpu.CompilerParams(needs_layout_passes=False,
                                             use_tc_tiling_on_sc=False),
    )(ctx_flat, u_pre, v16)


def _loss_body(s_ref, lab_ref, m_ref, out_ref):
    s = s_ref[...]
    lab = lab_ref[...]
    m = m_ref[...]
    per = jnp.maximum(s, 0.0) - s * lab + jnp.log1p(jnp.exp(-jnp.abs(s)))
    num = jnp.sum(per * m)
    den = jnp.maximum(jnp.sum(m), 1.0)
    out_ref[0, 0] = num / den


def _tc_loss(scores2d, label2d, mask2d):
    return pl.pallas_call(
        _loss_body,
        out_shape=jax.ShapeDtypeStruct((1, 1), jnp.float32),
        out_specs=pl.BlockSpec(memory_space=pltpu.SMEM),
    )(scores2d, label2d, mask2d)


def kernel(center, context_neg, label, mask, U, V):
    ctx_flat = context_neg.reshape(B * L)
    u_pre = jnp.take(U, center[:, 0], axis=0)
    scores = _sc_scores(ctx_flat, u_pre, V.astype(jnp.bfloat16))
    scores2d = scores.reshape(B * L // 128, 128)
    label2d = label.reshape(B * L // 128, 128)
    mask2d = mask.reshape(B * L // 128, 128)
    return _tc_loss(scores2d, label2d, mask2d).reshape(())
